# baseline (speedup 1.0000x reference)
import numpy as np

import jax
import jax.numpy as jnp
from jax import lax
from jax.experimental import pallas as pl
from jax.experimental.pallas import tpu as pltpu
from jax.experimental.pallas import tpu_sc as plsc

N = 10000
NP = 10240
E = 320000
D = 128

NC = 2
NS = 16
NW = NC * NS
EPW = E // NW
C = 80
NCHUNK = -(-EPW // C)
EPWP = NCHUNK * C
PAD = EPWP - EPW
KB = 2
DNB = 5
RPT = NP // NS
DW = 128

_mesh = plsc.VectorSubcoreMesh(core_axis_name="c", subcore_axis_name="s",
                               num_cores=NC, num_subcores=NS)


def _build_deg(interpret=False):
    return pl.kernel(
        _deg_body,
        out_type=(
            jax.ShapeDtypeStruct((NC, NP, DW), jnp.float32),
            jax.ShapeDtypeStruct((NC, NP, DW), jnp.float32),
        ),
        mesh=_mesh,
        scratch_types=[
            pltpu.VMEM((NCHUNK, C), jnp.int32),
            pltpu.VMEM((C, DW), jnp.float32),
            pltpu.VMEM_SHARED((NP, DW), jnp.float32),
            [pltpu.SemaphoreType.DMA] * DNB,
        ],
        interpret=interpret,
    )


def _deg_body(srcr_hbm, dstr_hbm, zcnt_hbm, ones_hbm, scnt_out, dcnt_out,
              idx_v, ones_v, cnt_sh, sems):
    c = lax.axis_index("c")
    s = lax.axis_index("s")
    wid = c * jnp.int32(NS) + s

    pltpu.sync_copy(ones_hbm, ones_v)
    row0 = s * jnp.int32(RPT)

    for e_hbm, out in ((srcr_hbm, scnt_out), (dstr_hbm, dcnt_out)):
        pltpu.sync_copy(e_hbm.at[wid], idx_v)
        pltpu.sync_copy(zcnt_hbm, cnt_sh.at[pl.ds(row0, RPT)])
        plsc.subcore_barrier()

        def group(_, j0):
            for b in range(DNB):
                pltpu.async_copy(ones_v, cnt_sh.at[idx_v.at[j0 + jnp.int32(b)]],
                                 sems[b], add=True)
            for b in range(DNB):
                pltpu.make_async_copy(ones_v, cnt_sh.at[idx_v.at[j0 + jnp.int32(b)]],
                                      sems[b]).wait()
            return j0 + jnp.int32(DNB)

        lax.fori_loop(0, NCHUNK // DNB, group, jnp.int32(0))
        plsc.subcore_barrier()

        pltpu.sync_copy(cnt_sh.at[pl.ds(row0, RPT)],
                        out.at[c, pl.ds(row0, RPT)])
        plsc.subcore_barrier()


def _build_agg(interpret=False):
    return pl.kernel(
        _agg_body,
        out_type=jax.ShapeDtypeStruct((NC, NP, D), jnp.float32),
        mesh=_mesh,
        scratch_types=[
            pltpu.VMEM((EPWP,), jnp.int32),
            pltpu.VMEM((NCHUNK, C), jnp.int32),
            pltpu.VMEM((KB, C, D), jnp.float32),
            pltpu.VMEM_SHARED((NP, D), jnp.float32),
            [pltpu.SemaphoreType.DMA] * (2 * KB),
        ],
        interpret=interpret,
    )


def _agg_body(h_hbm, srcf_hbm, dstr_hbm, zrow_hbm, out_hbm,
              sidx, didx, rows, agg_sh, sems):
    c = lax.axis_index("c")
    s = lax.axis_index("s")
    wid = c * jnp.int32(NS) + s

    row0 = s * jnp.int32(RPT)
    pltpu.sync_copy(zrow_hbm, agg_sh.at[pl.ds(row0, RPT)])
    pltpu.sync_copy(srcf_hbm.at[wid], sidx)
    pltpu.sync_copy(dstr_hbm.at[wid], didx)
    plsc.subcore_barrier()

    pltpu.async_copy(h_hbm.at[sidx.at[pl.ds(np.int32(0), C)]],
                     rows.at[np.int32(0)], sems[0]).wait()

    def group(_, carry):
        j0, off = carry
        o1 = pl.multiple_of(off + jnp.int32(C), 8)
        pltpu.async_copy(h_hbm.at[sidx.at[pl.ds(o1, C)]],
                         rows.at[np.int32(1)], sems[1])
        pltpu.async_copy(rows.at[np.int32(0)], agg_sh.at[didx.at[j0]],
                         sems[2], add=True)
        pltpu.make_async_copy(h_hbm.at[sidx.at[pl.ds(o1, C)]],
                              rows.at[np.int32(1)], sems[1]).wait()
        pltpu.make_async_copy(rows.at[np.int32(0)], agg_sh.at[didx.at[j0]],
                              sems[2]).wait()
        o2 = pl.multiple_of(off + jnp.int32(2 * C), 8)
        pltpu.async_copy(h_hbm.at[sidx.at[pl.ds(o2, C)]],
                         rows.at[np.int32(0)], sems[0])
        pltpu.async_copy(rows.at[np.int32(1)],
                         agg_sh.at[didx.at[j0 + jnp.int32(1)]],
                         sems[3], add=True)
        pltpu.make_async_copy(h_hbm.at[sidx.at[pl.ds(o2, C)]],
                              rows.at[np.int32(0)], sems[0]).wait()
        pltpu.make_async_copy(rows.at[np.int32(1)],
                              agg_sh.at[didx.at[j0 + jnp.int32(1)]],
                              sems[3]).wait()
        return (j0 + jnp.int32(2), off + jnp.int32(2 * C))

    lax.fori_loop(0, (NCHUNK - 1) // 2, group,
                  (jnp.int32(0), jnp.int32(0)))
    pltpu.sync_copy(rows.at[np.int32(0)],
                    agg_sh.at[didx.at[np.int32(NCHUNK - 1)]], add=True)
    plsc.subcore_barrier()

    pltpu.sync_copy(agg_sh.at[pl.ds(row0, RPT)],
                    out_hbm.at[c, pl.ds(row0, RPT)])


_RB = 1000
_RBP = 1024
_I0 = np.int32(0)


def _norm_prep_body(scnt_ref, dcnt_ref, feat_ref, ns_ref, nd_ref, out_ref):
    sdeg = scnt_ref[0, :, 0:1] + scnt_ref[1, :, 0:1]
    ddeg = dcnt_ref[0, :, 0:1] + dcnt_ref[1, :, 0:1]
    ns = lax.rsqrt(jnp.maximum(sdeg, 1.0))
    ns_ref[...] = ns
    nd_ref[...] = lax.rsqrt(jnp.maximum(ddeg, 1.0))
    out_ref[...] = feat_ref[...] * ns


def _norm_prep(scnt, dcnt, feat):
    return pl.pallas_call(
        _norm_prep_body,
        grid=(N // _RB,),
        in_specs=[
            pl.BlockSpec((NC, _RB, DW), lambda i: (_I0, i, _I0)),
            pl.BlockSpec((NC, _RB, DW), lambda i: (_I0, i, _I0)),
            pl.BlockSpec((_RB, D), lambda i: (i, _I0)),
        ],
        out_specs=[
            pl.BlockSpec((_RB, 1), lambda i: (i, _I0)),
            pl.BlockSpec((_RB, 1), lambda i: (i, _I0)),
            pl.BlockSpec((_RB, D), lambda i: (i, _I0)),
        ],
        out_shape=[
            jax.ShapeDtypeStruct((N, 1), jnp.float32),
            jax.ShapeDtypeStruct((N, 1), jnp.float32),
            jax.ShapeDtypeStruct((N, D), jnp.float32),
        ],
    )(scnt, dcnt, feat)


def _post_body(agg_ref, nd_ref, ns_ref, w_ref, b_ref,
               pre_ref, act_ref, nxt_ref):
    t = (agg_ref[0] + agg_ref[1]) * nd_ref[...]
    pre = jnp.dot(t, w_ref[...], preferred_element_type=jnp.float32) + b_ref[...]
    act = jnp.maximum(pre, 0.0)
    pre_ref[...] = pre
    act_ref[...] = act
    nxt_ref[...] = act * ns_ref[...]


def _post(aggp, nd, ns, W, b2d):
    return pl.pallas_call(
        _post_body,
        grid=(N // _RB,),
        in_specs=[
            pl.BlockSpec((NC, _RB, D), lambda i: (_I0, i, _I0)),
            pl.BlockSpec((_RB, 1), lambda i: (i, _I0)),
            pl.BlockSpec((_RB, 1), lambda i: (i, _I0)),
            pl.BlockSpec((D, D), lambda i: (_I0, _I0)),
            pl.BlockSpec((1, D), lambda i: (_I0, _I0)),
        ],
        out_specs=[
            pl.BlockSpec((_RB, D), lambda i: (i, _I0)),
            pl.BlockSpec((_RB, D), lambda i: (i, _I0)),
            pl.BlockSpec((_RB, D), lambda i: (i, _I0)),
        ],
        out_shape=[
            jax.ShapeDtypeStruct((N, D), jnp.float32),
            jax.ShapeDtypeStruct((N, D), jnp.float32),
            jax.ShapeDtypeStruct((N, D), jnp.float32),
        ],
    )(aggp, nd, ns, W, b2d)


_DEG = _build_deg()
_AGG = _build_agg()


def kernel(feat, edge_index, W1, b1, W2, b2, W3, b3):
    src_w = edge_index[0].astype(jnp.int32).reshape(NW, EPW)
    dst_w = edge_index[1].astype(jnp.int32).reshape(NW, EPW)
    dump = jnp.full((NW, PAD), NP - 1, jnp.int32)
    srcf = jnp.pad(src_w, ((0, 0), (0, PAD)))
    srcr = jnp.concatenate([src_w, dump], axis=1).reshape(NW, NCHUNK, C)
    dstr = jnp.concatenate([dst_w, dump], axis=1).reshape(NW, NCHUNK, C)
    feat = feat.astype(jnp.float32)
    zrow = jnp.zeros((RPT, D), jnp.float32)
    zcnt = jnp.zeros((RPT, DW), jnp.float32)
    ones = jnp.ones((C, DW), jnp.float32)

    scnt, dcnt = _DEG(srcr, dstr, zcnt, ones)
    ns, nd, h = _norm_prep(scnt, dcnt, feat)
    pres, acts = [], []
    for W, b in ((W1, b1), (W2, b2), (W3, b3)):
        aggp = _AGG(h, srcf, dstr, zrow)
        pre, act, h = _post(aggp, nd, ns, W.astype(jnp.float32),
                            b.astype(jnp.float32).reshape(1, D))
        pres.append(pre)
        acts.append(act)

    return (acts[2], feat, pres[0], pres[1], pres[2],
            feat, acts[0], acts[1], acts[2])


# speedup vs baseline: 1.0023x; 1.0023x over previous
import numpy as np

import jax
import jax.numpy as jnp
from jax import lax
from jax.experimental import pallas as pl
from jax.experimental.pallas import tpu as pltpu
from jax.experimental.pallas import tpu_sc as plsc

N = 10000
NP = 10240
E = 320000
D = 128

NC = 2
NS = 16
NW = NC * NS
EPW = E // NW
C = 80
NCHUNK = -(-EPW // C)
EPWP = NCHUNK * C
PAD = EPWP - EPW
KB = 2
DNB = 5
RPT = NP // NS
DW = 128

_mesh = plsc.VectorSubcoreMesh(core_axis_name="c", subcore_axis_name="s",
                               num_cores=NC, num_subcores=NS)


def _build_deg():
    return pl.kernel(
        _deg_body,
        out_type=(
            jax.ShapeDtypeStruct((NC, NP, DW), jnp.float32),
            jax.ShapeDtypeStruct((NC, NP, DW), jnp.float32),
        ),
        mesh=_mesh,
        scratch_types=[
            pltpu.VMEM((NCHUNK, C), jnp.int32),
            pltpu.VMEM((C, DW), jnp.float32),
            pltpu.VMEM_SHARED((NP, DW), jnp.float32),
            [pltpu.SemaphoreType.DMA] * DNB,
        ],
    )


def _deg_body(srcr_hbm, dstr_hbm, zcnt_hbm, ones_hbm, scnt_out, dcnt_out,
              idx_v, ones_v, cnt_sh, sems):
    c = lax.axis_index("c")
    s = lax.axis_index("s")
    wid = c * jnp.int32(NS) + s

    pltpu.sync_copy(ones_hbm, ones_v)
    row0 = s * jnp.int32(RPT)

    for e_hbm, out in ((srcr_hbm, scnt_out), (dstr_hbm, dcnt_out)):
        pltpu.sync_copy(e_hbm.at[wid], idx_v)
        pltpu.sync_copy(zcnt_hbm, cnt_sh.at[pl.ds(row0, RPT)])
        plsc.subcore_barrier()

        def group(_, j0):
            for b in range(DNB):
                pltpu.async_copy(ones_v, cnt_sh.at[idx_v.at[j0 + jnp.int32(b)]],
                                 sems[b], add=True)
            for b in range(DNB):
                pltpu.make_async_copy(ones_v, cnt_sh.at[idx_v.at[j0 + jnp.int32(b)]],
                                      sems[b]).wait()
            return j0 + jnp.int32(DNB)

        lax.fori_loop(0, NCHUNK // DNB, group, jnp.int32(0))
        plsc.subcore_barrier()

        pltpu.sync_copy(cnt_sh.at[pl.ds(row0, RPT)],
                        out.at[c, pl.ds(row0, RPT)])
        plsc.subcore_barrier()


def _build_agg():
    return pl.kernel(
        _agg_body,
        out_type=jax.ShapeDtypeStruct((NC, NP, D), jnp.float32),
        mesh=_mesh,
        scratch_types=[
            pltpu.VMEM((EPWP,), jnp.int32),
            pltpu.VMEM((NCHUNK, C), jnp.int32),
            pltpu.VMEM((KB, C, D), jnp.float32),
            pltpu.VMEM_SHARED((NP, D), jnp.float32),
            [pltpu.SemaphoreType.DMA] * KB,
        ],
    )


def _agg_body(h_hbm, srcf_hbm, dstr_hbm, zrow_hbm, out_hbm,
              sidx, didx, rows, agg_sh, sems):
    c = lax.axis_index("c")
    s = lax.axis_index("s")
    wid = c * jnp.int32(NS) + s

    row0 = s * jnp.int32(RPT)
    pltpu.sync_copy(zrow_hbm, agg_sh.at[pl.ds(row0, RPT)])
    pltpu.sync_copy(srcf_hbm.at[wid], sidx)
    pltpu.sync_copy(dstr_hbm.at[wid], didx)
    plsc.subcore_barrier()

    pltpu.async_copy(h_hbm.at[sidx.at[pl.ds(np.int32(0), C)]],
                     rows.at[np.int32(0)], sems[0]).wait()

    def group(_, carry):
        j0, off = carry
        o1 = pl.multiple_of(off + jnp.int32(C), 8)
        pltpu.async_copy(h_hbm.at[sidx.at[pl.ds(o1, C)]],
                         rows.at[np.int32(1)], sems[1])
        pltpu.sync_copy(rows.at[np.int32(0)], agg_sh.at[didx.at[j0]],
                        add=True)
        pltpu.make_async_copy(h_hbm.at[sidx.at[pl.ds(o1, C)]],
                              rows.at[np.int32(1)], sems[1]).wait()
        o2 = pl.multiple_of(off + jnp.int32(2 * C), 8)
        pltpu.async_copy(h_hbm.at[sidx.at[pl.ds(o2, C)]],
                         rows.at[np.int32(0)], sems[0])
        pltpu.sync_copy(rows.at[np.int32(1)],
                        agg_sh.at[didx.at[j0 + jnp.int32(1)]], add=True)
        pltpu.make_async_copy(h_hbm.at[sidx.at[pl.ds(o2, C)]],
                              rows.at[np.int32(0)], sems[0]).wait()
        return (j0 + jnp.int32(2), off + jnp.int32(2 * C))

    lax.fori_loop(0, (NCHUNK - 1) // 2, group,
                  (jnp.int32(0), jnp.int32(0)))
    pltpu.sync_copy(rows.at[np.int32(0)],
                    agg_sh.at[didx.at[np.int32(NCHUNK - 1)]], add=True)
    plsc.subcore_barrier()

    pltpu.sync_copy(agg_sh.at[pl.ds(row0, RPT)],
                    out_hbm.at[c, pl.ds(row0, RPT)])


_RB = 1000
_I0 = np.int32(0)


def _norm_prep_body(scnt_ref, dcnt_ref, feat_ref, ns_ref, nd_ref, out_ref):
    sdeg = scnt_ref[0, :, 0:1] + scnt_ref[1, :, 0:1]
    ddeg = dcnt_ref[0, :, 0:1] + dcnt_ref[1, :, 0:1]
    ns = lax.rsqrt(jnp.maximum(sdeg, 1.0))
    ns_ref[...] = ns
    nd_ref[...] = lax.rsqrt(jnp.maximum(ddeg, 1.0))
    out_ref[...] = feat_ref[...] * ns


def _norm_prep(scnt, dcnt, feat):
    return pl.pallas_call(
        _norm_prep_body,
        grid=(N // _RB,),
        in_specs=[
            pl.BlockSpec((NC, _RB, DW), lambda i: (_I0, i, _I0)),
            pl.BlockSpec((NC, _RB, DW), lambda i: (_I0, i, _I0)),
            pl.BlockSpec((_RB, D), lambda i: (i, _I0)),
        ],
        out_specs=[
            pl.BlockSpec((_RB, 1), lambda i: (i, _I0)),
            pl.BlockSpec((_RB, 1), lambda i: (i, _I0)),
            pl.BlockSpec((_RB, D), lambda i: (i, _I0)),
        ],
        out_shape=[
            jax.ShapeDtypeStruct((N, 1), jnp.float32),
            jax.ShapeDtypeStruct((N, 1), jnp.float32),
            jax.ShapeDtypeStruct((N, D), jnp.float32),
        ],
    )(scnt, dcnt, feat)


def _post_body(agg_ref, nd_ref, ns_ref, w_ref, b_ref,
               pre_ref, act_ref, nxt_ref):
    t = (agg_ref[0] + agg_ref[1]) * nd_ref[...]
    pre = jnp.dot(t, w_ref[...], preferred_element_type=jnp.float32) + b_ref[...]
    act = jnp.maximum(pre, 0.0)
    pre_ref[...] = pre
    act_ref[...] = act
    nxt_ref[...] = act * ns_ref[...]


def _post(aggp, nd, ns, W, b2d):
    return pl.pallas_call(
        _post_body,
        grid=(N // _RB,),
        in_specs=[
            pl.BlockSpec((NC, _RB, D), lambda i: (_I0, i, _I0)),
            pl.BlockSpec((_RB, 1), lambda i: (i, _I0)),
            pl.BlockSpec((_RB, 1), lambda i: (i, _I0)),
            pl.BlockSpec((D, D), lambda i: (_I0, _I0)),
            pl.BlockSpec((1, D), lambda i: (_I0, _I0)),
        ],
        out_specs=[
            pl.BlockSpec((_RB, D), lambda i: (i, _I0)),
            pl.BlockSpec((_RB, D), lambda i: (i, _I0)),
            pl.BlockSpec((_RB, D), lambda i: (i, _I0)),
        ],
        out_shape=[
            jax.ShapeDtypeStruct((N, D), jnp.float32),
            jax.ShapeDtypeStruct((N, D), jnp.float32),
            jax.ShapeDtypeStruct((N, D), jnp.float32),
        ],
    )(aggp, nd, ns, W, b2d)


_DEG = _build_deg()
_AGG = _build_agg()


def kernel(feat, edge_index, W1, b1, W2, b2, W3, b3):
    src_w = edge_index[0].astype(jnp.int32).reshape(NW, EPW)
    dst_w = edge_index[1].astype(jnp.int32).reshape(NW, EPW)
    dump = jnp.full((NW, PAD), NP - 1, jnp.int32)
    srcf = jnp.pad(src_w, ((0, 0), (0, PAD)))
    srcr = jnp.concatenate([src_w, dump], axis=1).reshape(NW, NCHUNK, C)
    dstr = jnp.concatenate([dst_w, dump], axis=1).reshape(NW, NCHUNK, C)
    feat = feat.astype(jnp.float32)
    zrow = jnp.zeros((RPT, D), jnp.float32)
    zcnt = jnp.zeros((RPT, DW), jnp.float32)
    ones = jnp.ones((C, DW), jnp.float32)

    scnt, dcnt = _DEG(srcr, dstr, zcnt, ones)
    ns, nd, h = _norm_prep(scnt, dcnt, feat)
    pres, acts = [], []
    for W, b in ((W1, b1), (W2, b2), (W3, b3)):
        aggp = _AGG(h, srcf, dstr, zrow)
        pre, act, h = _post(aggp, nd, ns, W.astype(jnp.float32),
                            b.astype(jnp.float32).reshape(1, D))
        pres.append(pre)
        acts.append(act)

    return (acts[2], feat, pres[0], pres[1], pres[2],
            feat, acts[0], acts[1], acts[2])
